# baseline (speedup 1.0000x reference)
import functools

import jax
import jax.numpy as jnp
from jax import lax
from jax.experimental import pallas as pl
from jax.experimental.pallas import tpu as pltpu
from jax.experimental.pallas import tpu_sc as plsc

BATCH = 16384
EMBED_DIM = 32
NUM_ROWS = 1000000
CHUNK = 128


def _make_kernel(num_cores, num_workers, b_per_w):
    mesh = plsc.VectorSubcoreMesh(core_axis_name="c", subcore_axis_name="s")
    n_md_chunks = b_per_w // CHUNK
    n_el = b_per_w * EMBED_DIM
    n_el_chunks = n_el // CHUNK

    @functools.partial(
        pl.kernel,
        mesh=mesh,
        compiler_params=pltpu.CompilerParams(needs_layout_passes=False),
        out_type=jax.ShapeDtypeStruct((BATCH,), jnp.float32),
        scratch_types=[
            pltpu.VMEM((b_per_w,), jnp.int32),
            pltpu.VMEM((b_per_w,), jnp.int32),
            pltpu.VMEM((n_el,), jnp.int32),
            pltpu.VMEM((n_el,), jnp.int32),
            pltpu.VMEM((n_el,), jnp.float32),
            pltpu.VMEM((n_el,), jnp.float32),
            pltpu.VMEM((b_per_w,), jnp.float32),
            pltpu.VMEM((b_per_w,), jnp.float32),
            pltpu.VMEM((b_per_w,), jnp.float32),
            pltpu.SemaphoreType.DMA,
            pltpu.SemaphoreType.DMA,
        ],
    )
    def sc_kernel(i_hbm, j_hbm, m_hbm, d_hbm, ut_hbm, vt_hbm, out_hbm,
                  idx_i, idx_j, idx_u, idx_v, u_vals, v_vals, m_v, d_v,
                  out_v, sem, sem_md):
        wid = lax.axis_index("s") * num_cores + lax.axis_index("c")
        base = wid * b_per_w

        pltpu.sync_copy(i_hbm.at[pl.ds(base, b_per_w)], idx_i)
        pltpu.sync_copy(j_hbm.at[pl.ds(base, b_per_w)], idx_j)

        md_copies = []
        for c in range(n_md_chunks):
            s = pl.ds(c * CHUNK, CHUNK)
            md_copies.append(
                pltpu.async_copy(m_hbm.at[idx_i.at[s]], m_v.at[s], sem_md))
            md_copies.append(
                pltpu.async_copy(d_hbm.at[idx_j.at[s]], d_v.at[s], sem_md))

        def build_body(g, carry):
            gb = g * 16
            iv = idx_i[pl.ds(gb, 16)]
            jv = idx_j[pl.ds(gb, 16)]
            for k in range(EMBED_DIM):
                off = k * NUM_ROWS
                idx_u[pl.ds(k * b_per_w + gb, 16)] = iv + off
                idx_v[pl.ds(k * b_per_w + gb, 16)] = jv + off
            return carry

        lax.fori_loop(0, b_per_w // 16, build_body, 0)

        for c in range(n_el_chunks):
            s = pl.ds(c * CHUNK, CHUNK)
            pltpu.async_copy(ut_hbm.at[idx_u.at[s]], u_vals.at[s], sem)
            pltpu.async_copy(vt_hbm.at[idx_v.at[s]], v_vals.at[s], sem)

        for cp in md_copies:
            cp.wait()
        pltpu.make_async_copy(
            ut_hbm.at[pl.ds(0, n_el)], u_vals, sem).wait()
        pltpu.make_async_copy(
            vt_hbm.at[pl.ds(0, n_el)], v_vals, sem).wait()

        def group_body(g, carry):
            gb = g * 16
            acc = m_v[pl.ds(gb, 16)] + d_v[pl.ds(gb, 16)]
            for k in range(EMBED_DIM):
                acc = acc + (u_vals[pl.ds(k * b_per_w + gb, 16)]
                             * v_vals[pl.ds(k * b_per_w + gb, 16)])
            out_v[pl.ds(gb, 16)] = acc
            return carry

        lax.fori_loop(0, b_per_w // 16, group_body, 0)

        pltpu.sync_copy(out_v, out_hbm.at[pl.ds(base, b_per_w)])

    return sc_kernel


def kernel(ij, m_bar, d_bar, U, V):
    i = jnp.asarray(ij[:, 0], dtype=jnp.int32)
    j = jnp.asarray(ij[:, 1], dtype=jnp.int32)
    info = plsc.get_sparse_core_info()
    num_workers = info.num_cores * info.num_subcores
    b_per_w = BATCH // num_workers
    ut = lax.optimization_barrier(U.T).reshape(-1)
    vt = lax.optimization_barrier(V.T).reshape(-1)
    return _make_kernel(info.num_cores, num_workers, b_per_w)(
        i, j, m_bar, d_bar, ut, vt)


# speedup vs baseline: 9.0131x; 9.0131x over previous
import functools

import jax
import jax.numpy as jnp
from jax import lax
from jax.experimental import pallas as pl
from jax.experimental.pallas import tpu as pltpu
from jax.experimental.pallas import tpu_sc as plsc

BATCH = 16384
EMBED_DIM = 32
NUM_ROWS = 1000000
CHUNK = 128
BAND_STRIDE = 8 * 1000064
N_FLAT = EMBED_DIM * NUM_ROWS
TAIL_K = 24
TAIL_I = 999808
TAIL_W = NUM_ROWS - TAIL_I
FCHUNK = 16384
N_FCHUNKS = 60
TCH_OFF = 999424
WINDOW = N_FLAT


def _make_flatten(num_cores):
    mesh = plsc.VectorSubcoreMesh(core_axis_name="c", subcore_axis_name="s")

    @functools.partial(
        pl.kernel,
        mesh=mesh,
        compiler_params=pltpu.CompilerParams(needs_layout_passes=False),
        out_type=(
            jax.ShapeDtypeStruct((N_FLAT,), jnp.float32),
            jax.ShapeDtypeStruct((N_FLAT,), jnp.float32),
        ),
        scratch_types=[
            pltpu.VMEM((FCHUNK,), jnp.float32),
            pltpu.VMEM((FCHUNK,), jnp.float32),
            pltpu.SemaphoreType.DMA,
            pltpu.SemaphoreType.DMA,
            pltpu.SemaphoreType.DMA,
            pltpu.SemaphoreType.DMA,
        ],
    )
    def flatten(ut_hbm, vt_hbm, xu_hbm, xv_hbm, buf_a, buf_b,
                semr_a, semw_a, semr_b, semw_b):
        wid = lax.axis_index("s") * num_cores + lax.axis_index("c")
        bufs = ((buf_a, semr_a, semw_a), (buf_b, semr_b, semw_b))

        def wait_bytes(ref, sem):
            pltpu.make_async_copy(
                ut_hbm.at[0, 0].at[pl.ds(0, FCHUNK)], ref, sem).wait()

        for job in range(2):
            g = job * 32 + wid
            tbl = g // 32
            rowflat = g % 32
            a = rowflat // 8
            r = rowflat % 8
            row_base = rowflat * NUM_ROWS

            def _off(c):
                return pl.multiple_of(c * FCHUNK, 128)

            def rd(c, buf, semr, tbl=tbl, a=a, r=r):
                src = pl.ds(_off(c), FCHUNK)

                @pl.when(tbl == 0)
                def _():
                    pltpu.async_copy(ut_hbm.at[a, r].at[src], buf, semr)

                @pl.when(tbl == 1)
                def _():
                    pltpu.async_copy(vt_hbm.at[a, r].at[src], buf, semr)

            def wr(c, buf, semw, tbl=tbl, row_base=row_base):
                dst = pl.ds(row_base + _off(c), FCHUNK)

                @pl.when(tbl == 0)
                def _():
                    pltpu.async_copy(buf, xu_hbm.at[dst], semw)

                @pl.when(tbl == 1)
                def _():
                    pltpu.async_copy(buf, xv_hbm.at[dst], semw)

            rd(0, buf_a, semr_a)
            rd(1, buf_b, semr_b)

            def body(c2, carry):
                for par in range(2):
                    buf, semr, semw = bufs[par]
                    c = c2 * 2 + par
                    wait_bytes(buf, semr)
                    wr(c, buf, semw)
                    nxt = c + 2

                    @pl.when(nxt < N_FCHUNKS)
                    def _(nxt=nxt, buf=buf, semr=semr, semw=semw):
                        wait_bytes(buf, semw)
                        rd(nxt, buf, semr)

                return carry

            lax.fori_loop(0, N_FCHUNKS // 2, body, 0)
            wait_bytes(buf_a, semw_a)
            wait_bytes(buf_b, semw_b)

            rd(60, buf_a, semr_a)
            wait_bytes(buf_a, semr_a)
            wr(60, buf_a, semw_a)
            wait_bytes(buf_a, semw_a)

            def rd512(buf, semr, tbl=tbl, a=a, r=r):
                src = pl.ds(TCH_OFF, 512)

                @pl.when(tbl == 0)
                def _():
                    pltpu.async_copy(
                        ut_hbm.at[a, r].at[src], buf.at[pl.ds(0, 512)], semr)

                @pl.when(tbl == 1)
                def _():
                    pltpu.async_copy(
                        vt_hbm.at[a, r].at[src], buf.at[pl.ds(0, 512)], semr)

            def wait512(buf, sem):
                pltpu.make_async_copy(
                    ut_hbm.at[0, 0].at[pl.ds(0, 512)],
                    buf.at[pl.ds(0, 512)], sem).wait()

            rd512(buf_a, semr_a)
            wait512(buf_a, semr_a)
            dst512 = pl.ds(row_base + TCH_OFF, 512)

            @pl.when(tbl == 0)
            def _(dst512=dst512):
                pltpu.async_copy(buf_a.at[pl.ds(0, 512)], xu_hbm.at[dst512],
                                 semw_a)

            @pl.when(tbl == 1)
            def _(dst512=dst512):
                pltpu.async_copy(buf_a.at[pl.ds(0, 512)], xv_hbm.at[dst512],
                                 semw_a)

            wait512(buf_a, semw_a)

    return flatten


def _make_gather(num_cores, num_workers, b_per_w):
    mesh = plsc.VectorSubcoreMesh(core_axis_name="c", subcore_axis_name="s")
    n_md_chunks = b_per_w // CHUNK
    n_el = b_per_w * EMBED_DIM
    n_el_chunks = n_el // CHUNK

    @functools.partial(
        pl.kernel,
        mesh=mesh,
        compiler_params=pltpu.CompilerParams(needs_layout_passes=False),
        out_type=jax.ShapeDtypeStruct((BATCH,), jnp.float32),
        scratch_types=[
            pltpu.VMEM((b_per_w,), jnp.int32),
            pltpu.VMEM((b_per_w,), jnp.int32),
            pltpu.VMEM((n_el,), jnp.int32),
            pltpu.VMEM((n_el,), jnp.int32),
            pltpu.VMEM((n_el,), jnp.float32),
            pltpu.VMEM((n_el,), jnp.float32),
            pltpu.VMEM((b_per_w,), jnp.float32),
            pltpu.VMEM((b_per_w,), jnp.float32),
            pltpu.VMEM((b_per_w,), jnp.float32),
            pltpu.VMEM((4096,), jnp.float32),
            pltpu.VMEM((4096,), jnp.float32),
            pltpu.SemaphoreType.DMA,
            pltpu.SemaphoreType.DMA,
        ],
    )
    def sc_kernel(i_hbm, j_hbm, m_hbm, d_hbm, xu_hbm, xv_hbm, tu_hbm, tv_hbm,
                  out_hbm, idx_i, idx_j, idx_u, idx_v, u_vals, v_vals,
                  m_v, d_v, out_v, tail_u, tail_v, sem, sem_md):
        wid = lax.axis_index("s") * num_cores + lax.axis_index("c")
        base = wid * b_per_w

        pltpu.sync_copy(i_hbm.at[pl.ds(base, b_per_w)], idx_i)
        pltpu.sync_copy(j_hbm.at[pl.ds(base, b_per_w)], idx_j)
        pltpu.sync_copy(tu_hbm, tail_u)
        pltpu.sync_copy(tv_hbm, tail_v)

        md_copies = []
        for c in range(n_md_chunks):
            s = pl.ds(c * CHUNK, CHUNK)
            md_copies.append(
                pltpu.async_copy(m_hbm.at[idx_i.at[s]], m_v.at[s], sem_md))
            md_copies.append(
                pltpu.async_copy(d_hbm.at[idx_j.at[s]], d_v.at[s], sem_md))

        def build_body(g, carry):
            gb = g * 16
            iv = idx_i[pl.ds(gb, 16)]
            jv = idx_j[pl.ds(gb, 16)]
            iw = ((iv >> 7) << 10) + (iv & 127)
            jw = ((jv >> 7) << 10) + (jv & 127)
            for k in range(EMBED_DIM):
                off = (k // 8) * BAND_STRIDE + (k % 8) * 128
                s = pl.ds(k * b_per_w + gb, 16)
                wu = iw + off
                wv = jw + off
                idx_u[s] = jnp.where(wu < WINDOW, wu, 0)
                idx_v[s] = jnp.where(wv < WINDOW, wv, 0)
            return carry

        lax.fori_loop(0, b_per_w // 16, build_body, 0)

        for c in range(n_el_chunks):
            s = pl.ds(c * CHUNK, CHUNK)
            pltpu.async_copy(xu_hbm.at[idx_u.at[s]], u_vals.at[s], sem)
            pltpu.async_copy(xv_hbm.at[idx_v.at[s]], v_vals.at[s], sem)

        for cp in md_copies:
            cp.wait()
        pltpu.make_async_copy(xu_hbm.at[pl.ds(0, n_el)], u_vals, sem).wait()
        pltpu.make_async_copy(xv_hbm.at[pl.ds(0, n_el)], v_vals, sem).wait()

        def patch(w, vals, tails):
            q = w // 1000000
            rem = w - q * 1000000
            c2 = rem >= TCH_OFF + 512
            c1 = w >= WINDOW
            tidx = jnp.where(
                c1, 2048 + (w - WINDOW),
                jnp.where(c2, (q << 6) + (rem - (TCH_OFF + 512)), 0))
            pv = plsc.load_gather(tails, [tidx])
            return jnp.where(jnp.logical_or(c1, c2), pv, vals)

        def group_body(g, carry):
            gb = g * 16
            iv = idx_i[pl.ds(gb, 16)]
            jv = idx_j[pl.ds(gb, 16)]
            iw = ((iv >> 7) << 10) + (iv & 127)
            jw = ((jv >> 7) << 10) + (jv & 127)
            acc = m_v[pl.ds(gb, 16)] + d_v[pl.ds(gb, 16)]
            for k in range(EMBED_DIM):
                off = (k // 8) * BAND_STRIDE + (k % 8) * 128
                uk = patch(iw + off,
                           u_vals[pl.ds(k * b_per_w + gb, 16)], tail_u)
                vk = patch(jw + off,
                           v_vals[pl.ds(k * b_per_w + gb, 16)], tail_v)
                acc = acc + uk * vk
            out_v[pl.ds(gb, 16)] = acc
            return carry

        lax.fori_loop(0, b_per_w // 16, group_body, 0)

        pltpu.sync_copy(out_v, out_hbm.at[pl.ds(base, b_per_w)])

    return sc_kernel


def kernel(ij, m_bar, d_bar, U, V):
    i = jnp.asarray(ij[:, 0], dtype=jnp.int32)
    j = jnp.asarray(ij[:, 1], dtype=jnp.int32)
    info = plsc.get_sparse_core_info()
    num_workers = info.num_cores * info.num_subcores
    b_per_w = BATCH // num_workers

    ut4 = U.T.reshape(4, 8, NUM_ROWS)
    vt4 = V.T.reshape(4, 8, NUM_ROWS)
    xu, xv = _make_flatten(info.num_cores)(ut4, vt4)

    def tails_of(w4, wt):
        t2 = w4[:, :, TCH_OFF + 512:].reshape(-1)
        pad = jnp.pad(wt[TAIL_K:, TAIL_I:], ((0, 0), (0, 64)))
        t1 = pad.reshape(8, 2, 128).transpose(1, 0, 2).reshape(-1)
        return jnp.concatenate([t2, t1])

    tails_u = tails_of(ut4, U.T)
    tails_v = tails_of(vt4, V.T)
    return _make_gather(info.num_cores, num_workers, b_per_w)(
        i, j, m_bar, d_bar, xu, xv, tails_u, tails_v)


# speedup vs baseline: 9.0674x; 1.0060x over previous
import functools

import jax
import jax.numpy as jnp
from jax import lax
from jax.experimental import pallas as pl
from jax.experimental.pallas import tpu as pltpu
from jax.experimental.pallas import tpu_sc as plsc

BATCH = 16384
EMBED_DIM = 32
NUM_ROWS = 1000000
CHUNK = 128
BAND_STRIDE = 8 * 1000064
N_FLAT = EMBED_DIM * NUM_ROWS
TAIL_K = 24
TAIL_I = 999808
TAIL_W = NUM_ROWS - TAIL_I
FCHUNK = 16384
N_FCHUNKS = 60
TCH_OFF = 999424
WINDOW = N_FLAT


def _make_flatten(num_cores):
    mesh = plsc.VectorSubcoreMesh(core_axis_name="c", subcore_axis_name="s")
    NBUF = 4
    N_FULL = 2 * 61

    @functools.partial(
        pl.kernel,
        mesh=mesh,
        compiler_params=pltpu.CompilerParams(needs_layout_passes=False),
        out_type=(
            jax.ShapeDtypeStruct((N_FLAT,), jnp.float32),
            jax.ShapeDtypeStruct((N_FLAT,), jnp.float32),
        ),
        scratch_types=(
            [pltpu.VMEM((FCHUNK,), jnp.float32)] * 4
            + [pltpu.SemaphoreType.DMA] * 8
        ),
    )
    def flatten(ut_hbm, vt_hbm, xu_hbm, xv_hbm,
                buf_0, buf_1, buf_2, buf_3,
                semr_0, semr_1, semr_2, semr_3,
                semw_0, semw_1, semw_2, semw_3):
        wid = lax.axis_index("s") * num_cores + lax.axis_index("c")
        bufs = ((buf_0, semr_0, semw_0), (buf_1, semr_1, semw_1),
                (buf_2, semr_2, semw_2), (buf_3, semr_3, semw_3))

        def wait_bytes(ref, sem):
            pltpu.make_async_copy(
                ut_hbm.at[0, 0].at[pl.ds(0, FCHUNK)], ref, sem).wait()

        def parts(cglob):
            job = cglob // 61
            c = cglob % 61
            g = job * 32 + wid
            tbl = g // 32
            rowflat = g % 32
            return tbl, rowflat // 8, rowflat % 8, rowflat * NUM_ROWS, c

        def rd(cglob, buf, semr):
            tbl, a, r, row_base, c = parts(cglob)
            src = pl.ds(pl.multiple_of(c * FCHUNK, 128), FCHUNK)

            @pl.when(tbl == 0)
            def _():
                pltpu.async_copy(ut_hbm.at[a, r].at[src], buf, semr)

            @pl.when(tbl == 1)
            def _():
                pltpu.async_copy(vt_hbm.at[a, r].at[src], buf, semr)

        def wr(cglob, buf, semw):
            tbl, a, r, row_base, c = parts(cglob)
            dst = pl.ds(
                pl.multiple_of(row_base + c * FCHUNK, 128), FCHUNK)

            @pl.when(tbl == 0)
            def _():
                pltpu.async_copy(buf, xu_hbm.at[dst], semw)

            @pl.when(tbl == 1)
            def _():
                pltpu.async_copy(buf, xv_hbm.at[dst], semw)

        for par in range(NBUF):
            rd(par, bufs[par][0], bufs[par][1])

        def body(t, carry):
            for par in range(NBUF):
                buf, semr, semw = bufs[par]
                cglob = t * NBUF + par

                @pl.when(cglob < N_FULL)
                def _(cglob=cglob, buf=buf, semr=semr, semw=semw):
                    wait_bytes(buf, semr)
                    wr(cglob, buf, semw)

                nxt = cglob + NBUF

                @pl.when(nxt < N_FULL)
                def _(nxt=nxt, buf=buf, semr=semr, semw=semw):
                    wait_bytes(buf, semw)
                    rd(nxt, buf, semr)

            return carry

        lax.fori_loop(0, (N_FULL + NBUF - 1) // NBUF, body, 0)
        for par in range(NBUF):
            wait_bytes(bufs[par][0], bufs[par][2])

        def rd512(tbl, a, r, buf, semr):
            src = pl.ds(TCH_OFF, 512)

            @pl.when(tbl == 0)
            def _():
                pltpu.async_copy(
                    ut_hbm.at[a, r].at[src], buf.at[pl.ds(0, 512)], semr)

            @pl.when(tbl == 1)
            def _():
                pltpu.async_copy(
                    vt_hbm.at[a, r].at[src], buf.at[pl.ds(0, 512)], semr)

        def wait512(buf, sem):
            pltpu.make_async_copy(
                ut_hbm.at[0, 0].at[pl.ds(0, 512)],
                buf.at[pl.ds(0, 512)], sem).wait()

        for job in range(2):
            g = job * 32 + wid
            tbl = g // 32
            rowflat = g % 32
            a = rowflat // 8
            r = rowflat % 8
            buf, semr, semw = bufs[job]
            rd512(tbl, a, r, buf, semr)
            wait512(buf, semr)
            dst512 = pl.ds(rowflat * NUM_ROWS + TCH_OFF, 512)

            @pl.when(tbl == 0)
            def _(dst512=dst512, buf=buf, semw=semw):
                pltpu.async_copy(buf.at[pl.ds(0, 512)], xu_hbm.at[dst512],
                                 semw)

            @pl.when(tbl == 1)
            def _(dst512=dst512, buf=buf, semw=semw):
                pltpu.async_copy(buf.at[pl.ds(0, 512)], xv_hbm.at[dst512],
                                 semw)

            wait512(buf, semw)

    return flatten


def _make_gather(num_cores, num_workers, b_per_w):
    mesh = plsc.VectorSubcoreMesh(core_axis_name="c", subcore_axis_name="s")
    n_md_chunks = b_per_w // CHUNK
    n_el = b_per_w * EMBED_DIM
    n_el_chunks = n_el // CHUNK

    @functools.partial(
        pl.kernel,
        mesh=mesh,
        compiler_params=pltpu.CompilerParams(needs_layout_passes=False),
        out_type=jax.ShapeDtypeStruct((BATCH,), jnp.float32),
        scratch_types=[
            pltpu.VMEM((b_per_w,), jnp.int32),
            pltpu.VMEM((b_per_w,), jnp.int32),
            pltpu.VMEM((n_el,), jnp.int32),
            pltpu.VMEM((n_el,), jnp.int32),
            pltpu.VMEM((n_el,), jnp.float32),
            pltpu.VMEM((n_el,), jnp.float32),
            pltpu.VMEM((b_per_w,), jnp.float32),
            pltpu.VMEM((b_per_w,), jnp.float32),
            pltpu.VMEM((b_per_w,), jnp.float32),
            pltpu.VMEM((4096,), jnp.float32),
            pltpu.VMEM((4096,), jnp.float32),
            pltpu.SemaphoreType.DMA,
            pltpu.SemaphoreType.DMA,
        ],
    )
    def sc_kernel(i_hbm, j_hbm, m_hbm, d_hbm, xu_hbm, xv_hbm, tu_hbm, tv_hbm,
                  out_hbm, idx_i, idx_j, idx_u, idx_v, u_vals, v_vals,
                  m_v, d_v, out_v, tail_u, tail_v, sem, sem_md):
        wid = lax.axis_index("s") * num_cores + lax.axis_index("c")
        base = wid * b_per_w

        pltpu.sync_copy(i_hbm.at[pl.ds(base, b_per_w)], idx_i)
        pltpu.sync_copy(j_hbm.at[pl.ds(base, b_per_w)], idx_j)
        pltpu.sync_copy(tu_hbm, tail_u)
        pltpu.sync_copy(tv_hbm, tail_v)

        md_copies = []
        for c in range(n_md_chunks):
            s = pl.ds(c * CHUNK, CHUNK)
            md_copies.append(
                pltpu.async_copy(m_hbm.at[idx_i.at[s]], m_v.at[s], sem_md))
            md_copies.append(
                pltpu.async_copy(d_hbm.at[idx_j.at[s]], d_v.at[s], sem_md))

        def build_body(g, carry):
            gb = g * 16
            iv = idx_i[pl.ds(gb, 16)]
            jv = idx_j[pl.ds(gb, 16)]
            iw = ((iv >> 7) << 10) + (iv & 127)
            jw = ((jv >> 7) << 10) + (jv & 127)
            for k in range(EMBED_DIM):
                off = (k // 8) * BAND_STRIDE + (k % 8) * 128
                s = pl.ds(k * b_per_w + gb, 16)
                wu = iw + off
                wv = jw + off
                idx_u[s] = jnp.where(wu < WINDOW, wu, 0)
                idx_v[s] = jnp.where(wv < WINDOW, wv, 0)
            return carry

        lax.fori_loop(0, b_per_w // 16, build_body, 0)

        for c in range(n_el_chunks):
            s = pl.ds(c * CHUNK, CHUNK)
            pltpu.async_copy(xu_hbm.at[idx_u.at[s]], u_vals.at[s], sem)
            pltpu.async_copy(xv_hbm.at[idx_v.at[s]], v_vals.at[s], sem)

        for cp in md_copies:
            cp.wait()
        pltpu.make_async_copy(xu_hbm.at[pl.ds(0, n_el)], u_vals, sem).wait()
        pltpu.make_async_copy(xv_hbm.at[pl.ds(0, n_el)], v_vals, sem).wait()

        def patch(w, vals, tails):
            q = w // 1000000
            rem = w - q * 1000000
            c2 = rem >= TCH_OFF + 512
            c1 = w >= WINDOW
            tidx = jnp.where(
                c1, 2048 + (w - WINDOW),
                jnp.where(c2, (q << 6) + (rem - (TCH_OFF + 512)), 0))
            pv = plsc.load_gather(tails, [tidx])
            return jnp.where(jnp.logical_or(c1, c2), pv, vals)

        def group_body(g, carry):
            gb = g * 16
            iv = idx_i[pl.ds(gb, 16)]
            jv = idx_j[pl.ds(gb, 16)]
            iw = ((iv >> 7) << 10) + (iv & 127)
            jw = ((jv >> 7) << 10) + (jv & 127)
            acc = m_v[pl.ds(gb, 16)] + d_v[pl.ds(gb, 16)]
            for k in range(EMBED_DIM):
                off = (k // 8) * BAND_STRIDE + (k % 8) * 128
                uk = patch(iw + off,
                           u_vals[pl.ds(k * b_per_w + gb, 16)], tail_u)
                vk = patch(jw + off,
                           v_vals[pl.ds(k * b_per_w + gb, 16)], tail_v)
                acc = acc + uk * vk
            out_v[pl.ds(gb, 16)] = acc
            return carry

        lax.fori_loop(0, b_per_w // 16, group_body, 0)

        pltpu.sync_copy(out_v, out_hbm.at[pl.ds(base, b_per_w)])

    return sc_kernel


def kernel(ij, m_bar, d_bar, U, V):
    i = jnp.asarray(ij[:, 0], dtype=jnp.int32)
    j = jnp.asarray(ij[:, 1], dtype=jnp.int32)
    info = plsc.get_sparse_core_info()
    num_workers = info.num_cores * info.num_subcores
    b_per_w = BATCH // num_workers

    ut4 = U.T.reshape(4, 8, NUM_ROWS)
    vt4 = V.T.reshape(4, 8, NUM_ROWS)
    xu, xv = _make_flatten(info.num_cores)(ut4, vt4)

    def tails_of(w4, wt):
        t2 = w4[:, :, TCH_OFF + 512:].reshape(-1)
        pad = jnp.pad(wt[TAIL_K:, TAIL_I:], ((0, 0), (0, 64)))
        t1 = pad.reshape(8, 2, 128).transpose(1, 0, 2).reshape(-1)
        return jnp.concatenate([t2, t1])

    tails_u = tails_of(ut4, U.T)
    tails_v = tails_of(vt4, V.T)
    return _make_gather(info.num_cores, num_workers, b_per_w)(
        i, j, m_bar, d_bar, xu, xv, tails_u, tails_v)
